# baseline (speedup 1.0000x reference)
import functools

import jax
import jax.numpy as jnp
from jax import lax
from jax.experimental import pallas as pl
from jax.experimental.pallas import tpu as pltpu
from jax.experimental.pallas import tpu_sc as plsc

N = 10000
D = 128
E = 320000

NC, NS = 2, 16
NW = NC * NS
CHUNK = 128
CPW = 80
E_PAD = NW * CPW * CHUNK
NBUF = 2
ACC_ROWS = 10112
ZROWS = ACC_ROWS // NS
STAGE = 16
NSTAGE = CPW // STAGE

@functools.cache
def _build_seg_sum():
    mesh = plsc.VectorSubcoreMesh(core_axis_name="c", subcore_axis_name="s",
                                  num_cores=NC, num_subcores=NS)
    return functools.partial(
        pl.kernel,
        out_type=jax.ShapeDtypeStruct((NC * ACC_ROWS, D), jnp.float32),
        mesh=mesh,
        scratch_types=_seg_scratch(),
    )(_seg_sum_body)


def _seg_scratch():
    return ([
        pltpu.VMEM_SHARED((ACC_ROWS, D), jnp.float32),
        pltpu.VMEM((STAGE, CHUNK), jnp.int32),
        pltpu.VMEM((STAGE, CHUNK), jnp.int32),
    ] + [pltpu.VMEM((CHUNK, D), jnp.float32)] * NBUF
      + [pltpu.SemaphoreType.DMA] * (2 * NBUF)
      + [pltpu.SemaphoreType.DMA])


def _seg_sum(h, src2d, dst2d):
    return _build_seg_sum()(h, src2d, dst2d)


@functools.cache
def _build_seg_sum2():
    mesh = plsc.VectorSubcoreMesh(core_axis_name="c", subcore_axis_name="s",
                                  num_cores=NC, num_subcores=NS)
    return functools.partial(
        pl.kernel,
        out_type=jax.ShapeDtypeStruct((2 * NC * ACC_ROWS, D), jnp.float32),
        mesh=mesh,
        scratch_types=_seg_scratch(),
    )(_seg_sum2_body)


def _seg_sum2(x, h, src2d, dst2d):
    return _build_seg_sum2()(x, h, src2d, dst2d)


def _zero_acc_slice(zb, acc, z_base, sem):
    def _zb(i, carry):
        zb[i // 8, pl.ds((i % 8) * 16, 16)] = jnp.zeros((16,), jnp.float32)
        return carry
    lax.fori_loop(0, CHUNK * D // 16, _zb, 0)
    nz = ZROWS // CHUNK
    rem = ZROWS % CHUNK
    zcps = []
    for j in range(nz):
        zcps.append(pltpu.async_copy(
            zb, acc.at[pl.ds(z_base + j * CHUNK, CHUNK)], sem))
    if rem:
        zcps.append(pltpu.async_copy(
            zb.at[pl.ds(0, rem)],
            acc.at[pl.ds(z_base + nz * CHUNK, rem)], sem))
    for cp in zcps:
        cp.wait()


def _gather_scatter_half(tbl, acc, srcv, dstv, rows, semg, sems):
    for j in range(NBUF - 1):
        pltpu.async_copy(tbl.at[srcv.at[j]], rows[j], semg[j])

    def _body(i, carry):
        base = NBUF * i
        for j in range(NBUF):
            ci = base + j
            bj = (j + NBUF - 1) % NBUF
            pltpu.make_async_copy(tbl.at[srcv.at[ci]], rows[j],
                                  semg[j]).wait()
            pltpu.async_copy(rows[j], acc.at[dstv.at[ci]], sems[j],
                             add=True)
            nxt = ci + NBUF - 1
            if j == 0:
                @pl.when(base >= 1)
                def _():
                    pltpu.make_async_copy(rows[bj], acc.at[dstv.at[0]],
                                          sems[bj]).wait()
                pltpu.async_copy(tbl.at[srcv.at[nxt]], rows[bj], semg[bj])
            else:
                @pl.when(nxt < STAGE)
                def _():
                    pltpu.make_async_copy(rows[bj], acc.at[dstv.at[0]],
                                          sems[bj]).wait()
                    pltpu.async_copy(tbl.at[srcv.at[nxt]], rows[bj],
                                     semg[bj])
        return carry
    lax.fori_loop(0, STAGE // NBUF, _body, 0)
    for j in range(NBUF):
        pltpu.make_async_copy(rows[j], acc.at[dstv.at[0]], sems[j]).wait()


def _stage_idx(src_hbm, dst_hbm, srcv, dstv, idx_base, st):
    pltpu.sync_copy(src_hbm.at[pl.ds(idx_base + st * STAGE, STAGE)], srcv)
    pltpu.sync_copy(dst_hbm.at[pl.ds(idx_base + st * STAGE, STAGE)], dstv)


def _seg_sum2_body(x_hbm, h_hbm, src_hbm, dst_hbm, out_hbm, acc, srcv, dstv,
                   *rs):
    rows, semg, sems, semi = rs[:NBUF], rs[NBUF:2 * NBUF], \
        rs[2 * NBUF:3 * NBUF], rs[3 * NBUF]
    c = lax.axis_index("c")
    s = lax.axis_index("s")
    w = c * NS + s
    idx_base = pl.multiple_of(w * CPW, 8)
    z_base = pl.multiple_of(s * ZROWS, 8)

    cp_s = pltpu.async_copy(src_hbm.at[pl.ds(idx_base, STAGE)], srcv, semi)
    cp_d = pltpu.async_copy(dst_hbm.at[pl.ds(idx_base, STAGE)], dstv, semi)
    _zero_acc_slice(rows[0], acc, z_base, semg[0])
    cp_s.wait()
    cp_d.wait()
    plsc.subcore_barrier()

    staged = 0
    for t, tbl in ((0, x_hbm), (1, h_hbm)):
        for st in (range(NSTAGE) if t == 0 else range(NSTAGE - 1, -1, -1)):
            if st != staged:
                _stage_idx(src_hbm, dst_hbm, srcv, dstv, idx_base, st)
                staged = st
            _gather_scatter_half(tbl, acc, srcv, dstv, rows, semg, sems)
        plsc.subcore_barrier()
        o_base = pl.multiple_of(
            t * NC * ACC_ROWS + c * ACC_ROWS + s * ZROWS, 8)
        pltpu.sync_copy(acc.at[pl.ds(z_base, ZROWS)],
                        out_hbm.at[pl.ds(o_base, ZROWS)])
        if t == 0:
            _zero_acc_slice(rows[0], acc, z_base, semg[0])
            plsc.subcore_barrier()


def _seg_sum_body(h_hbm, src_hbm, dst_hbm, out_hbm, acc, srcv, dstv, *rs):
    rows, semg, sems, semi = rs[:NBUF], rs[NBUF:2 * NBUF], \
        rs[2 * NBUF:3 * NBUF], rs[3 * NBUF]
    c = lax.axis_index("c")
    s = lax.axis_index("s")
    w = c * NS + s
    idx_base = pl.multiple_of(w * CPW, 8)
    z_base = pl.multiple_of(s * ZROWS, 8)
    o_base = pl.multiple_of(c * ACC_ROWS + s * ZROWS, 8)

    cp_s = pltpu.async_copy(src_hbm.at[pl.ds(idx_base, STAGE)], srcv, semi)
    cp_d = pltpu.async_copy(dst_hbm.at[pl.ds(idx_base, STAGE)], dstv, semi)
    _zero_acc_slice(rows[0], acc, z_base, semg[0])
    cp_s.wait()
    cp_d.wait()
    plsc.subcore_barrier()

    for st in range(NSTAGE):
        if st != 0:
            _stage_idx(src_hbm, dst_hbm, srcv, dstv, idx_base, st)
        _gather_scatter_half(h_hbm, acc, srcv, dstv, rows, semg, sems)

    plsc.subcore_barrier()
    pltpu.sync_copy(acc.at[pl.ds(z_base, ZROWS)],
                    out_hbm.at[pl.ds(o_base, ZROWS)])


_BS = 1000


def _dot(a, b):
    return lax.dot_general(a, b, (((1,), (1,)), ((), ())),
                           preferred_element_type=jnp.float32)


def _lin_in_body(x, w, b, o):
    o[...] = _dot(x[...], w[...]) + b[...]


def _mlp_body(x, ax0, ax1, h, ah0, ah1, w1x, w1h, b1, g, be, w2, b2, w3, b3,
              o):
    zx = x[...] + ax0[...] + ax1[...]
    zh = h[...] + ah0[...] + ah1[...]
    u = _dot(zx, w1x[...]) + _dot(zh, w1h[...]) + b1[...]
    mu = jnp.mean(u, axis=-1, keepdims=True)
    var = jnp.mean((u - mu) ** 2, axis=-1, keepdims=True)
    u = (u - mu) / jnp.sqrt(var + 1e-5) * g[...] + be[...]
    u = jnp.where(u > 0, u, 0.1 * u)
    v = _dot(u, w2[...]) + b2[...]
    v = jnp.where(v > 0, v, 0.1 * v)
    o[...] = _dot(v, w3[...]) + b3[...]


def _mlp_out_body(x, ax0, ax1, h, ah0, ah1, h0, w1x, w1h, b1, g, be, w2, b2,
                  w3, b3, wo0, wo1, wo2, bo, o):
    zx = x[...] + ax0[...] + ax1[...]
    zh = h[...] + ah0[...] + ah1[...]
    u = _dot(zx, w1x[...]) + _dot(zh, w1h[...]) + b1[...]
    mu = jnp.mean(u, axis=-1, keepdims=True)
    var = jnp.mean((u - mu) ** 2, axis=-1, keepdims=True)
    u = (u - mu) / jnp.sqrt(var + 1e-5) * g[...] + be[...]
    u = jnp.where(u > 0, u, 0.1 * u)
    v = _dot(u, w2[...]) + b2[...]
    v = jnp.where(v > 0, v, 0.1 * v)
    h2 = _dot(v, w3[...]) + b3[...]
    o[...] = (_dot(h0[...], wo0[...]) + _dot(h[...], wo1[...])
              + _dot(h2, wo2[...]) + bo[...])


def _row_spec():
    return pl.BlockSpec((_BS, D), lambda i: (i, 0))


def _full_spec(shape):
    return pl.BlockSpec(shape, lambda i: tuple(0 for _ in shape))


def _lin_in(x, w, b):
    return pl.pallas_call(
        _lin_in_body,
        grid=(N // _BS,),
        in_specs=[_row_spec(), _full_spec(w.shape), _full_spec(b.shape)],
        out_specs=_row_spec(),
        out_shape=jax.ShapeDtypeStruct((N, D), jnp.float32),
    )(x, w, b)


def _mlp(x, ax0, ax1, h, ah0, ah1, p):
    w1x, w1h = p["W1"][:, :D], p["W1"][:, D:]
    args = (x, ax0, ax1, h, ah0, ah1, w1x, w1h, p["b1"].reshape(1, -1),
            p["g"].reshape(1, -1), p["be"].reshape(1, -1), p["W2"],
            p["b2"].reshape(1, -1), p["W3"], p["b3"].reshape(1, -1))
    specs = [_row_spec()] * 6 + [_full_spec(a.shape) for a in args[6:]]
    return pl.pallas_call(
        _mlp_body,
        grid=(N // _BS,),
        in_specs=specs,
        out_specs=_row_spec(),
        out_shape=jax.ShapeDtypeStruct((N, D), jnp.float32),
    )(*args)


def _mlp_out(x, ax0, ax1, h, ah0, ah1, h0, p, wo0, wo1, wo2, bo):
    w1x, w1h = p["W1"][:, :D], p["W1"][:, D:]
    args = (x, ax0, ax1, h, ah0, ah1, h0, w1x, w1h, p["b1"].reshape(1, -1),
            p["g"].reshape(1, -1), p["be"].reshape(1, -1), p["W2"],
            p["b2"].reshape(1, -1), p["W3"], p["b3"].reshape(1, -1),
            wo0, wo1, wo2, bo.reshape(1, -1))
    specs = [_row_spec()] * 7 + [_full_spec(a.shape) for a in args[7:]]
    return pl.pallas_call(
        _mlp_out_body,
        grid=(N // _BS,),
        in_specs=specs,
        out_specs=_row_spec(),
        out_shape=jax.ShapeDtypeStruct((N, D), jnp.float32),
    )(*args)


def kernel(x, edge_index, params):
    src = edge_index[0]
    dst = edge_index[1]
    pad = E_PAD - E
    ar = jnp.arange(pad, dtype=jnp.int32)
    src_p = jnp.concatenate([src, ar % N])
    dst_p = jnp.concatenate([dst, N + ar % (ACC_ROWS - N)])
    src2d = src_p.reshape(NW * CPW, CHUNK)
    dst2d = dst_p.reshape(NW * CPW, CHUNK)

    h0 = _lin_in(x, params["in_W"], params["in_b"].reshape(1, -1))
    agg2 = _seg_sum2(x, h0, src2d, dst2d)

    def parts(agg):
        return agg[:N], agg[ACC_ROWS:ACC_ROWS + N]

    p0, p1 = params["convs"][0], params["convs"][1]
    ax0, ax1 = parts(agg2)
    ah0, ah1 = parts(agg2[2 * ACC_ROWS:])
    h1 = _mlp(x, ax0, ax1, h0, ah0, ah1, p0)
    aggh1 = _seg_sum(h1, src2d, dst2d)
    a10, a11 = parts(aggh1)

    wo = params["out_W"]
    out = _mlp_out(x, ax0, ax1, h1, a10, a11, h0, p1,
                   wo[:, :D], wo[:, D:2 * D], wo[:, 2 * D:],
                   params["out_b"])
    return out


# speedup vs baseline: 1.1955x; 1.1955x over previous
import functools

import jax
import jax.numpy as jnp
from jax import lax
from jax.experimental import pallas as pl
from jax.experimental.pallas import tpu as pltpu
from jax.experimental.pallas import tpu_sc as plsc

N = 10000
D = 128
E = 320000

NC, NS = 2, 16
NW = NC * NS
CHUNK = 128
CPW = 80
E_PAD = NW * CPW * CHUNK
NBUF = 2
ACC_ROWS = 10112
ZROWS = ACC_ROWS // NS
HALF = CPW // 2

@functools.cache
def _build_seg_sum():
    mesh = plsc.VectorSubcoreMesh(core_axis_name="c", subcore_axis_name="s",
                                  num_cores=NC, num_subcores=NS)
    return functools.partial(
        pl.kernel,
        out_type=jax.ShapeDtypeStruct((NC * ACC_ROWS, D), jnp.float32),
        mesh=mesh,
        scratch_types=_seg_scratch(),
    )(_seg_sum_body)


def _seg_scratch():
    return ([
        pltpu.VMEM_SHARED((ACC_ROWS, D), jnp.float32),
        pltpu.VMEM((HALF, CHUNK), jnp.int32),
        pltpu.VMEM((HALF, CHUNK), jnp.int32),
    ] + [pltpu.VMEM((CHUNK, D), jnp.float32)] * 2
      + [pltpu.SemaphoreType.DMA] * 3)


def _seg_sum(h, src2d, dst2d):
    return _build_seg_sum()(h, src2d, dst2d)


@functools.cache
def _build_seg_sum2():
    mesh = plsc.VectorSubcoreMesh(core_axis_name="c", subcore_axis_name="s",
                                  num_cores=NC, num_subcores=NS)
    return functools.partial(
        pl.kernel,
        out_type=jax.ShapeDtypeStruct((2 * NC * ACC_ROWS, D), jnp.float32),
        mesh=mesh,
        scratch_types=_seg_scratch(),
    )(_seg_sum2_body)


def _seg_sum2(x, h, src2d, dst2d):
    return _build_seg_sum2()(x, h, src2d, dst2d)


def _zero_acc_slice(zb, acc, z_base, sem):
    def _zb(i, carry):
        zb[i // 8, pl.ds((i % 8) * 16, 16)] = jnp.zeros((16,), jnp.float32)
        return carry
    lax.fori_loop(0, CHUNK * D // 16, _zb, 0)
    nz = ZROWS // CHUNK
    rem = ZROWS % CHUNK
    zcps = []
    for j in range(nz):
        zcps.append(pltpu.async_copy(
            zb, acc.at[pl.ds(z_base + j * CHUNK, CHUNK)], sem))
    if rem:
        zcps.append(pltpu.async_copy(
            zb.at[pl.ds(0, rem)],
            acc.at[pl.ds(z_base + nz * CHUNK, rem)], sem))
    for cp in zcps:
        cp.wait()


def _gather_scatter_half(tbl, acc, srcv, dstv, rows_a, rows_b, sem_a, sem_b):
    pltpu.async_copy(tbl.at[srcv.at[0]], rows_a, sem_a)

    def _body(i, carry):
        ci = 2 * i
        pltpu.async_copy(tbl.at[srcv.at[ci + 1]], rows_b, sem_b)
        pltpu.make_async_copy(tbl.at[srcv.at[ci]], rows_a, sem_a).wait()
        pltpu.sync_copy(rows_a, acc.at[dstv.at[ci]], add=True)

        @pl.when(ci + 2 < HALF)
        def _():
            pltpu.async_copy(tbl.at[srcv.at[ci + 2]], rows_a, sem_a)

        pltpu.make_async_copy(tbl.at[srcv.at[ci + 1]], rows_b, sem_b).wait()
        pltpu.sync_copy(rows_b, acc.at[dstv.at[ci + 1]], add=True)
        return carry
    lax.fori_loop(0, HALF // 2, _body, 0)


def _stage_idx(src_hbm, dst_hbm, srcv, dstv, idx_base, hf):
    pltpu.sync_copy(src_hbm.at[pl.ds(idx_base + hf * HALF, HALF)], srcv)
    pltpu.sync_copy(dst_hbm.at[pl.ds(idx_base + hf * HALF, HALF)], dstv)


def _seg_sum2_body(x_hbm, h_hbm, src_hbm, dst_hbm, out_hbm, acc, srcv, dstv,
                   rows_a, rows_b, sem_a, sem_b, semi):
    c = lax.axis_index("c")
    s = lax.axis_index("s")
    w = c * NS + s
    idx_base = pl.multiple_of(w * CPW, 8)
    z_base = pl.multiple_of(s * ZROWS, 8)

    cp_s = pltpu.async_copy(src_hbm.at[pl.ds(idx_base, HALF)], srcv, semi)
    cp_d = pltpu.async_copy(dst_hbm.at[pl.ds(idx_base, HALF)], dstv, semi)
    _zero_acc_slice(rows_a, acc, z_base, sem_a)
    cp_s.wait()
    cp_d.wait()
    plsc.subcore_barrier()

    staged = 0
    for t, tbl in ((0, x_hbm), (1, h_hbm)):
        for hf in ((0, 1) if t == 0 else (1, 0)):
            if hf != staged:
                _stage_idx(src_hbm, dst_hbm, srcv, dstv, idx_base, hf)
                staged = hf
            _gather_scatter_half(tbl, acc, srcv, dstv, rows_a, rows_b,
                                 sem_a, sem_b)
        plsc.subcore_barrier()
        o_base = pl.multiple_of(
            t * NC * ACC_ROWS + c * ACC_ROWS + s * ZROWS, 8)
        pltpu.sync_copy(acc.at[pl.ds(z_base, ZROWS)],
                        out_hbm.at[pl.ds(o_base, ZROWS)])
        if t == 0:
            _zero_acc_slice(rows_a, acc, z_base, sem_a)
            plsc.subcore_barrier()


def _seg_sum_body(h_hbm, src_hbm, dst_hbm, out_hbm, acc, srcv, dstv, rows_a,
                  rows_b, sem_a, sem_b, semi):
    c = lax.axis_index("c")
    s = lax.axis_index("s")
    w = c * NS + s
    idx_base = pl.multiple_of(w * CPW, 8)
    z_base = pl.multiple_of(s * ZROWS, 8)
    o_base = pl.multiple_of(c * ACC_ROWS + s * ZROWS, 8)

    cp_s = pltpu.async_copy(src_hbm.at[pl.ds(idx_base, HALF)], srcv, semi)
    cp_d = pltpu.async_copy(dst_hbm.at[pl.ds(idx_base, HALF)], dstv, semi)
    _zero_acc_slice(rows_a, acc, z_base, sem_a)
    cp_s.wait()
    cp_d.wait()
    plsc.subcore_barrier()

    for hf in range(2):
        if hf == 1:
            _stage_idx(src_hbm, dst_hbm, srcv, dstv, idx_base, hf)
        _gather_scatter_half(h_hbm, acc, srcv, dstv, rows_a, rows_b,
                             sem_a, sem_b)

    plsc.subcore_barrier()
    pltpu.sync_copy(acc.at[pl.ds(z_base, ZROWS)],
                    out_hbm.at[pl.ds(o_base, ZROWS)])


_BS = 1000


def _dot(a, b):
    return lax.dot_general(a, b, (((1,), (1,)), ((), ())),
                           preferred_element_type=jnp.float32)


def _lin_in_body(x, w, b, o):
    o[...] = _dot(x[...], w[...]) + b[...]


def _mlp_body(x, ax0, ax1, h, ah0, ah1, w1x, w1h, b1, g, be, w2, b2, w3, b3,
              o):
    zx = x[...] + ax0[...] + ax1[...]
    zh = h[...] + ah0[...] + ah1[...]
    u = _dot(zx, w1x[...]) + _dot(zh, w1h[...]) + b1[...]
    mu = jnp.mean(u, axis=-1, keepdims=True)
    var = jnp.mean((u - mu) ** 2, axis=-1, keepdims=True)
    u = (u - mu) / jnp.sqrt(var + 1e-5) * g[...] + be[...]
    u = jnp.where(u > 0, u, 0.1 * u)
    v = _dot(u, w2[...]) + b2[...]
    v = jnp.where(v > 0, v, 0.1 * v)
    o[...] = _dot(v, w3[...]) + b3[...]


def _mlp_out_body(x, ax0, ax1, h, ah0, ah1, h0, w1x, w1h, b1, g, be, w2, b2,
                  w3, b3, wo0, wo1, wo2, bo, o):
    zx = x[...] + ax0[...] + ax1[...]
    zh = h[...] + ah0[...] + ah1[...]
    u = _dot(zx, w1x[...]) + _dot(zh, w1h[...]) + b1[...]
    mu = jnp.mean(u, axis=-1, keepdims=True)
    var = jnp.mean((u - mu) ** 2, axis=-1, keepdims=True)
    u = (u - mu) / jnp.sqrt(var + 1e-5) * g[...] + be[...]
    u = jnp.where(u > 0, u, 0.1 * u)
    v = _dot(u, w2[...]) + b2[...]
    v = jnp.where(v > 0, v, 0.1 * v)
    h2 = _dot(v, w3[...]) + b3[...]
    o[...] = (_dot(h0[...], wo0[...]) + _dot(h[...], wo1[...])
              + _dot(h2, wo2[...]) + bo[...])


def _row_spec():
    return pl.BlockSpec((_BS, D), lambda i: (i, 0))


def _full_spec(shape):
    return pl.BlockSpec(shape, lambda i: tuple(0 for _ in shape))


def _lin_in(x, w, b):
    return pl.pallas_call(
        _lin_in_body,
        grid=(N // _BS,),
        in_specs=[_row_spec(), _full_spec(w.shape), _full_spec(b.shape)],
        out_specs=_row_spec(),
        out_shape=jax.ShapeDtypeStruct((N, D), jnp.float32),
    )(x, w, b)


def _mlp(x, ax0, ax1, h, ah0, ah1, p):
    w1x, w1h = p["W1"][:, :D], p["W1"][:, D:]
    args = (x, ax0, ax1, h, ah0, ah1, w1x, w1h, p["b1"].reshape(1, -1),
            p["g"].reshape(1, -1), p["be"].reshape(1, -1), p["W2"],
            p["b2"].reshape(1, -1), p["W3"], p["b3"].reshape(1, -1))
    specs = [_row_spec()] * 6 + [_full_spec(a.shape) for a in args[6:]]
    return pl.pallas_call(
        _mlp_body,
        grid=(N // _BS,),
        in_specs=specs,
        out_specs=_row_spec(),
        out_shape=jax.ShapeDtypeStruct((N, D), jnp.float32),
    )(*args)


def _mlp_out(x, ax0, ax1, h, ah0, ah1, h0, p, wo0, wo1, wo2, bo):
    w1x, w1h = p["W1"][:, :D], p["W1"][:, D:]
    args = (x, ax0, ax1, h, ah0, ah1, h0, w1x, w1h, p["b1"].reshape(1, -1),
            p["g"].reshape(1, -1), p["be"].reshape(1, -1), p["W2"],
            p["b2"].reshape(1, -1), p["W3"], p["b3"].reshape(1, -1),
            wo0, wo1, wo2, bo.reshape(1, -1))
    specs = [_row_spec()] * 7 + [_full_spec(a.shape) for a in args[7:]]
    return pl.pallas_call(
        _mlp_out_body,
        grid=(N // _BS,),
        in_specs=specs,
        out_specs=_row_spec(),
        out_shape=jax.ShapeDtypeStruct((N, D), jnp.float32),
    )(*args)


def kernel(x, edge_index, params):
    src = edge_index[0]
    dst = edge_index[1]
    pad = E_PAD - E
    ar = jnp.arange(pad, dtype=jnp.int32)
    src_p = jnp.concatenate([src, ar % N])
    dst_p = jnp.concatenate([dst, N + ar % (ACC_ROWS - N)])
    src2d = src_p.reshape(NW * CPW, CHUNK)
    dst2d = dst_p.reshape(NW * CPW, CHUNK)

    h0 = _lin_in(x, params["in_W"], params["in_b"].reshape(1, -1))
    agg2 = _seg_sum2(x, h0, src2d, dst2d)

    def parts(agg):
        return agg[:N], agg[ACC_ROWS:ACC_ROWS + N]

    p0, p1 = params["convs"][0], params["convs"][1]
    ax0, ax1 = parts(agg2)
    ah0, ah1 = parts(agg2[2 * ACC_ROWS:])
    h1 = _mlp(x, ax0, ax1, h0, ah0, ah1, p0)
    aggh1 = _seg_sum(h1, src2d, dst2d)
    a10, a11 = parts(aggh1)

    wo = params["out_W"]
    out = _mlp_out(x, ax0, ax1, h1, a10, a11, h0, p1,
                   wo[:, :D], wo[:, D:2 * D], wo[:, 2 * D:],
                   params["out_b"])
    return out


# speedup vs baseline: 1.2169x; 1.0179x over previous
import functools

import jax
import jax.numpy as jnp
from jax import lax
from jax.experimental import pallas as pl
from jax.experimental.pallas import tpu as pltpu
from jax.experimental.pallas import tpu_sc as plsc

N = 10000
D = 128
E = 320000

NC, NS = 2, 16
NW = NC * NS
CHUNK = 128
CPW = 80
E_PAD = NW * CPW * CHUNK
NBUF = 2
ACC_ROWS = 10112
ZROWS = ACC_ROWS // NS
HALF = CPW // 2

@functools.cache
def _build_seg_sum():
    mesh = plsc.VectorSubcoreMesh(core_axis_name="c", subcore_axis_name="s",
                                  num_cores=NC, num_subcores=NS)
    return functools.partial(
        pl.kernel,
        out_type=jax.ShapeDtypeStruct((NC * ACC_ROWS, D), jnp.float32),
        mesh=mesh,
        scratch_types=_seg_scratch(),
    )(_seg_sum_body)


def _seg_scratch():
    return ([
        pltpu.VMEM_SHARED((ACC_ROWS, D), jnp.float32),
        pltpu.VMEM((HALF, CHUNK), jnp.int32),
        pltpu.VMEM((HALF, CHUNK), jnp.int32),
    ] + [pltpu.VMEM((CHUNK, D), jnp.float32)] * 2
      + [pltpu.SemaphoreType.DMA] * 3)


def _seg_sum(h, src2d, dst2d):
    return _build_seg_sum()(h, src2d, dst2d)


@functools.cache
def _build_seg_sum2():
    mesh = plsc.VectorSubcoreMesh(core_axis_name="c", subcore_axis_name="s",
                                  num_cores=NC, num_subcores=NS)
    return functools.partial(
        pl.kernel,
        out_type=jax.ShapeDtypeStruct((2 * NC * ACC_ROWS, D), jnp.float32),
        mesh=mesh,
        scratch_types=_seg_scratch(),
    )(_seg_sum2_body)


def _seg_sum2(x, h, src2d, dst2d):
    return _build_seg_sum2()(x, h, src2d, dst2d)


def _zero_acc_slice(zb, acc, z_base, sem):
    def _zb(i, carry):
        zb[i // 8, pl.ds((i % 8) * 16, 16)] = jnp.zeros((16,), jnp.float32)
        return carry
    lax.fori_loop(0, CHUNK * D // 16, _zb, 0)
    nz = ZROWS // CHUNK
    rem = ZROWS % CHUNK
    zcps = []
    for j in range(nz):
        zcps.append(pltpu.async_copy(
            zb, acc.at[pl.ds(z_base + j * CHUNK, CHUNK)], sem))
    if rem:
        zcps.append(pltpu.async_copy(
            zb.at[pl.ds(0, rem)],
            acc.at[pl.ds(z_base + nz * CHUNK, rem)], sem))
    for cp in zcps:
        cp.wait()


def _gather_scatter_half(tbl, acc, srcv, dstv, rows_a, rows_b, sem_a, sem_b):
    pltpu.async_copy(tbl.at[srcv.at[0]], rows_a, sem_a)

    def _body(i, carry):
        ci = 2 * i
        pltpu.async_copy(tbl.at[srcv.at[ci + 1]], rows_b, sem_b)
        pltpu.make_async_copy(tbl.at[srcv.at[ci]], rows_a, sem_a).wait()
        pltpu.sync_copy(rows_a, acc.at[dstv.at[ci]], add=True)

        @pl.when(ci + 2 < HALF)
        def _():
            pltpu.async_copy(tbl.at[srcv.at[ci + 2]], rows_a, sem_a)

        pltpu.make_async_copy(tbl.at[srcv.at[ci + 1]], rows_b, sem_b).wait()
        pltpu.sync_copy(rows_b, acc.at[dstv.at[ci + 1]], add=True)
        return carry
    lax.fori_loop(0, HALF // 2, _body, 0)


def _stage_idx(src_hbm, dst_hbm, srcv, dstv, idx_base, hf):
    pltpu.sync_copy(src_hbm.at[pl.ds(idx_base + hf * HALF, HALF)], srcv)
    pltpu.sync_copy(dst_hbm.at[pl.ds(idx_base + hf * HALF, HALF)], dstv)


def _seg_sum2_body(x_hbm, h_hbm, src_hbm, dst_hbm, out_hbm, acc, srcv, dstv,
                   rows_a, rows_b, sem_a, sem_b, semi):
    c = lax.axis_index("c")
    s = lax.axis_index("s")
    w = c * NS + s
    idx_base = pl.multiple_of(w * CPW, 8)
    z_base = pl.multiple_of(s * ZROWS, 8)

    cp_s = pltpu.async_copy(src_hbm.at[pl.ds(idx_base, HALF)], srcv, semi)
    cp_d = pltpu.async_copy(dst_hbm.at[pl.ds(idx_base, HALF)], dstv, semi)
    _zero_acc_slice(rows_a, acc, z_base, sem_a)
    cp_s.wait()
    cp_d.wait()
    plsc.subcore_barrier()

    staged = 0
    for t, tbl in ((0, x_hbm), (1, h_hbm)):
        for hf in ((0, 1) if t == 0 else (1, 0)):
            if hf != staged:
                _stage_idx(src_hbm, dst_hbm, srcv, dstv, idx_base, hf)
                staged = hf
            _gather_scatter_half(tbl, acc, srcv, dstv, rows_a, rows_b,
                                 sem_a, sem_b)
        plsc.subcore_barrier()
        o_base = pl.multiple_of(
            t * NC * ACC_ROWS + c * ACC_ROWS + s * ZROWS, 8)
        pltpu.sync_copy(acc.at[pl.ds(z_base, ZROWS)],
                        out_hbm.at[pl.ds(o_base, ZROWS)])
        if t == 0:
            _zero_acc_slice(rows_a, acc, z_base, sem_a)
            plsc.subcore_barrier()


def _seg_sum_body(h_hbm, src_hbm, dst_hbm, out_hbm, acc, srcv, dstv, rows_a,
                  rows_b, sem_a, sem_b, semi):
    c = lax.axis_index("c")
    s = lax.axis_index("s")
    w = c * NS + s
    idx_base = pl.multiple_of(w * CPW, 8)
    z_base = pl.multiple_of(s * ZROWS, 8)
    o_base = pl.multiple_of(c * ACC_ROWS + s * ZROWS, 8)

    cp_s = pltpu.async_copy(src_hbm.at[pl.ds(idx_base, HALF)], srcv, semi)
    cp_d = pltpu.async_copy(dst_hbm.at[pl.ds(idx_base, HALF)], dstv, semi)
    _zero_acc_slice(rows_a, acc, z_base, sem_a)
    cp_s.wait()
    cp_d.wait()
    plsc.subcore_barrier()

    for hf in range(2):
        if hf == 1:
            _stage_idx(src_hbm, dst_hbm, srcv, dstv, idx_base, hf)
        _gather_scatter_half(h_hbm, acc, srcv, dstv, rows_a, rows_b,
                             sem_a, sem_b)

    plsc.subcore_barrier()
    pltpu.sync_copy(acc.at[pl.ds(z_base, ZROWS)],
                    out_hbm.at[pl.ds(o_base, ZROWS)])


_BS = 2000


def _dot(a, b):
    return lax.dot_general(a, b, (((1,), (1,)), ((), ())),
                           preferred_element_type=jnp.float32)


def _lin_in_body(x, w, b, o):
    o[...] = _dot(x[...], w[...]) + b[...]


def _mlp_body(x, ax0, ax1, h, ah0, ah1, w1x, w1h, b1, g, be, w2, b2, w3, b3,
              o):
    zx = x[...] + ax0[...] + ax1[...]
    zh = h[...] + ah0[...] + ah1[...]
    u = _dot(zx, w1x[...]) + _dot(zh, w1h[...]) + b1[...]
    mu = jnp.mean(u, axis=-1, keepdims=True)
    var = jnp.mean((u - mu) ** 2, axis=-1, keepdims=True)
    u = (u - mu) / jnp.sqrt(var + 1e-5) * g[...] + be[...]
    u = jnp.where(u > 0, u, 0.1 * u)
    v = _dot(u, w2[...]) + b2[...]
    v = jnp.where(v > 0, v, 0.1 * v)
    o[...] = _dot(v, w3[...]) + b3[...]


def _mlp_out_body(x, ax0, ax1, h, ah0, ah1, h0, w1x, w1h, b1, g, be, w2, b2,
                  w3, b3, wo0, wo1, wo2, bo, o):
    zx = x[...] + ax0[...] + ax1[...]
    zh = h[...] + ah0[...] + ah1[...]
    u = _dot(zx, w1x[...]) + _dot(zh, w1h[...]) + b1[...]
    mu = jnp.mean(u, axis=-1, keepdims=True)
    var = jnp.mean((u - mu) ** 2, axis=-1, keepdims=True)
    u = (u - mu) / jnp.sqrt(var + 1e-5) * g[...] + be[...]
    u = jnp.where(u > 0, u, 0.1 * u)
    v = _dot(u, w2[...]) + b2[...]
    v = jnp.where(v > 0, v, 0.1 * v)
    h2 = _dot(v, w3[...]) + b3[...]
    o[...] = (_dot(h0[...], wo0[...]) + _dot(h[...], wo1[...])
              + _dot(h2, wo2[...]) + bo[...])


def _row_spec():
    return pl.BlockSpec((_BS, D), lambda i: (i, 0))


def _full_spec(shape):
    return pl.BlockSpec(shape, lambda i: tuple(0 for _ in shape))


def _lin_in(x, w, b):
    return pl.pallas_call(
        _lin_in_body,
        grid=(N // _BS,),
        in_specs=[_row_spec(), _full_spec(w.shape), _full_spec(b.shape)],
        out_specs=_row_spec(),
        out_shape=jax.ShapeDtypeStruct((N, D), jnp.float32),
    )(x, w, b)


def _mlp(x, ax0, ax1, h, ah0, ah1, p):
    w1x, w1h = p["W1"][:, :D], p["W1"][:, D:]
    args = (x, ax0, ax1, h, ah0, ah1, w1x, w1h, p["b1"].reshape(1, -1),
            p["g"].reshape(1, -1), p["be"].reshape(1, -1), p["W2"],
            p["b2"].reshape(1, -1), p["W3"], p["b3"].reshape(1, -1))
    specs = [_row_spec()] * 6 + [_full_spec(a.shape) for a in args[6:]]
    return pl.pallas_call(
        _mlp_body,
        grid=(N // _BS,),
        in_specs=specs,
        out_specs=_row_spec(),
        out_shape=jax.ShapeDtypeStruct((N, D), jnp.float32),
    )(*args)


def _mlp_out(x, ax0, ax1, h, ah0, ah1, h0, p, wo0, wo1, wo2, bo):
    w1x, w1h = p["W1"][:, :D], p["W1"][:, D:]
    args = (x, ax0, ax1, h, ah0, ah1, h0, w1x, w1h, p["b1"].reshape(1, -1),
            p["g"].reshape(1, -1), p["be"].reshape(1, -1), p["W2"],
            p["b2"].reshape(1, -1), p["W3"], p["b3"].reshape(1, -1),
            wo0, wo1, wo2, bo.reshape(1, -1))
    specs = [_row_spec()] * 7 + [_full_spec(a.shape) for a in args[7:]]
    return pl.pallas_call(
        _mlp_out_body,
        grid=(N // _BS,),
        in_specs=specs,
        out_specs=_row_spec(),
        out_shape=jax.ShapeDtypeStruct((N, D), jnp.float32),
    )(*args)


def kernel(x, edge_index, params):
    src = edge_index[0]
    dst = edge_index[1]
    pad = E_PAD - E
    ar = jnp.arange(pad, dtype=jnp.int32)
    src_p = jnp.concatenate([src, ar % N])
    dst_p = jnp.concatenate([dst, N + ar % (ACC_ROWS - N)])
    src2d = src_p.reshape(NW * CPW, CHUNK)
    dst2d = dst_p.reshape(NW * CPW, CHUNK)

    h0 = _lin_in(x, params["in_W"], params["in_b"].reshape(1, -1))
    agg2 = _seg_sum2(x, h0, src2d, dst2d)

    def parts(agg):
        return agg[:N], agg[ACC_ROWS:ACC_ROWS + N]

    p0, p1 = params["convs"][0], params["convs"][1]
    ax0, ax1 = parts(agg2)
    ah0, ah1 = parts(agg2[2 * ACC_ROWS:])
    h1 = _mlp(x, ax0, ax1, h0, ah0, ah1, p0)
    aggh1 = _seg_sum(h1, src2d, dst2d)
    a10, a11 = parts(aggh1)

    wo = params["out_W"]
    out = _mlp_out(x, ax0, ax1, h1, a10, a11, h0, p1,
                   wo[:, :D], wo[:, D:2 * D], wo[:, 2 * D:],
                   params["out_b"])
    return out
